# baseline (speedup 1.0000x reference)
import functools

import jax
import jax.numpy as jnp
from jax.experimental import pallas as pl
from jax.experimental.pallas import tpu as pltpu

EMBED = 64
NUM_ROWS = 5


def _tc_body(city_ref, table_ref, out_ref):
    c = city_ref[...]
    r = c.shape[0]
    cb = jnp.broadcast_to(c, (r, EMBED))
    acc = jnp.broadcast_to(table_ref[0:1, :], (r, EMBED))
    for row in range(1, NUM_ROWS):
        acc = jnp.where(cb == row, jnp.broadcast_to(table_ref[row:row + 1, :], (r, EMBED)), acc)
    out_ref[...] = acc


@functools.partial(jax.jit, static_argnames=("block_r",))
def _tc_embed(table, idx_flat, block_r):
    b = idx_flat.shape[0]
    nb = b // block_r
    city2 = idx_flat.reshape(b, 1)
    return pl.pallas_call(
        _tc_body,
        grid=(nb,),
        in_specs=[
            pl.BlockSpec((block_r, 1), lambda i: (i, 0)),
            pl.BlockSpec((NUM_ROWS, EMBED), lambda i: (0, 0)),
        ],
        out_specs=pl.BlockSpec((block_r, EMBED), lambda i: (i, 0)),
        out_shape=jax.ShapeDtypeStruct((b, EMBED), jnp.float32),
        compiler_params=pltpu.CompilerParams(
            dimension_semantics=("arbitrary",)),
    )(city2, table)


def kernel(city, table):
    b0, b1 = city.shape
    idx_flat = city.reshape(b0 * b1)
    out = _tc_embed(table, idx_flat, 2048)
    return out.reshape(b0, b1, EMBED)
